# baseline (speedup 1.0000x reference)
import functools

import jax
import jax.numpy as jnp
from jax import lax
from jax.experimental import pallas as pl
from jax.experimental.pallas import tpu as pltpu
from jax.experimental.pallas import tpu_sc as plsc

L = 16
NTILES = 32
CONF = 16384
ATOMS = 200
ROWS_PER_TILE = CONF // NTILES
QSTEPS = ATOMS // 4
NCHUNKS = 4
CHUNK_ROWS = ROWS_PER_TILE // NCHUNKS
CHUNK_GROUPS = CHUNK_ROWS // L
CHUNK_WORDS = CHUNK_ROWS * ATOMS


def _sae_body(spec_hbm, en_hbm, se_hbm, out_hbm,
              buf_a, buf_b, t8_v, t2_v, t4_v, en_v, out_v, sem_a, sem_b):
    c = lax.axis_index("c")
    s = lax.axis_index("s")
    wid = s * 2 + c
    base = wid * ROWS_PER_TILE
    wbase = base * ATOMS

    bufs = (buf_a, buf_b)
    sems = (sem_a, sem_b)

    def start(k):
        return pltpu.async_copy(
            spec_hbm.at[pl.ds(wbase + k * CHUNK_WORDS, CHUNK_WORDS)],
            bufs[k % 2], sems[k % 2])

    pending = [start(0), start(1)]

    pltpu.sync_copy(se_hbm, t8_v)
    pltpu.sync_copy(en_hbm.at[pl.ds(base, ROWS_PER_TILE)], en_v)

    iota = lax.iota(jnp.int32, L)

    for v in range(4):
        idx = iota + v * L
        t2_v[pl.ds(v * L, L)] = (plsc.load_gather(t8_v, [(idx >> 3) & 7])
                                 + plsc.load_gather(t8_v, [idx & 7]))

    def build(v, carry):
        b0 = v * (8 * L)
        for u in range(8):
            idx = iota + (b0 + u * L)
            t4_v[pl.ds(b0 + u * L, L)] = (plsc.load_gather(t2_v, [idx >> 6])
                                          + plsc.load_gather(t2_v, [idx & 63]))
        return carry

    lax.fori_loop(0, 4096 // (8 * L), build, 0)

    one = jnp.full((L,), 1, jnp.int32)

    for k in range(NCHUNKS):
        buf = bufs[k % 2]
        pending[k % 2].wait()

        def group(g, carry, buf=buf, k=k):
            idx = (iota + g * L) * ATOMS
            acc_a = en_v[pl.ds(k * CHUNK_ROWS + g * L, L)]
            acc_b = jnp.zeros((L,), jnp.float32)
            for j in range(QSTEPS):
                s0 = plsc.load_gather(buf, [idx])
                idx = idx + one
                s1 = plsc.load_gather(buf, [idx])
                idx = idx + one
                s2 = plsc.load_gather(buf, [idx])
                idx = idx + one
                s3 = plsc.load_gather(buf, [idx])
                if j != QSTEPS - 1:
                    idx = idx + one
                pidx = (s0 << 9) | (s1 << 6) | (s2 << 3) | s3
                val = plsc.load_gather(t4_v, [pidx])
                if j % 2 == 0:
                    acc_a = acc_a + val
                else:
                    acc_b = acc_b + val
            out_v[pl.ds(k * CHUNK_ROWS + g * L, L)] = acc_a + acc_b
            return carry

        lax.fori_loop(0, CHUNK_GROUPS, group, 0)

        if k + 2 < NCHUNKS:
            pending[k % 2] = start(k + 2)

    pltpu.sync_copy(out_v, out_hbm.at[pl.ds(base, ROWS_PER_TILE)])


def _make_sae():
    mesh = plsc.VectorSubcoreMesh(core_axis_name="c", subcore_axis_name="s")
    return functools.partial(
        pl.kernel,
        mesh=mesh,
        compiler_params=pltpu.CompilerParams(needs_layout_passes=False),
        out_type=jax.ShapeDtypeStruct((CONF,), jnp.float32),
        scratch_types=[
            pltpu.VMEM((CHUNK_WORDS,), jnp.int32),
            pltpu.VMEM((CHUNK_WORDS,), jnp.int32),
            pltpu.VMEM((8,), jnp.float32),
            pltpu.VMEM((64,), jnp.float32),
            pltpu.VMEM((4096,), jnp.float32),
            pltpu.VMEM((ROWS_PER_TILE,), jnp.float32),
            pltpu.VMEM((ROWS_PER_TILE,), jnp.float32),
            pltpu.SemaphoreType.DMA,
            pltpu.SemaphoreType.DMA,
        ],
    )(_sae_body)


def kernel(species, energies, self_energies):
    spec_flat = species.astype(jnp.int32).reshape(CONF * ATOMS)
    se8 = jnp.zeros((8,), jnp.float32).at[:7].set(self_energies.astype(jnp.float32))
    out = _make_sae()(spec_flat, energies.astype(jnp.float32), se8)
    return (species, out)


# speedup vs baseline: 1.4773x; 1.4773x over previous
import functools

import jax
import jax.numpy as jnp
from jax import lax
from jax.experimental import pallas as pl
from jax.experimental.pallas import tpu as pltpu
from jax.experimental.pallas import tpu_sc as plsc

L = 16
NTILES = 32
CONF = 16384
ATOMS = 200
ROWS_PER_TILE = CONF // NTILES
NCHUNKS = 4
CHUNK_ROWS = ROWS_PER_TILE // NCHUNKS
CHUNK_WORDS = CHUNK_ROWS * ATOMS
SG_ROWS = 16
SGS_PER_CHUNK = CHUNK_ROWS // SG_ROWS


def _sae_body(spec_hbm, en_hbm, se_hbm, out_hbm,
              buf_a, buf_b, t8_v, t8r_v, t2_v, t4_v, red_v, en_v, out_v,
              sem_a, sem_b):
    c = lax.axis_index("c")
    s = lax.axis_index("s")
    wid = s * 2 + c
    base = wid * ROWS_PER_TILE

    bufs = (buf_a, buf_b)
    sems = (sem_a, sem_b)

    def start(k):
        return pltpu.async_copy(
            spec_hbm.at[pl.ds(base + k * CHUNK_ROWS, CHUNK_ROWS)],
            bufs[k % 2], sems[k % 2])

    pending = [start(0), start(1)]

    pltpu.sync_copy(se_hbm, t8_v)
    pltpu.sync_copy(en_hbm.at[pl.ds(base, ROWS_PER_TILE)], en_v)

    iota = lax.iota(jnp.int32, L)
    i17 = iota * 17
    i16 = iota * 16
    sel_lo = iota < 8
    zf = jnp.zeros((L,), jnp.float32)

    for v in range(4):
        idx = iota + v * L
        t2_v[pl.ds(v * L, L)] = (plsc.load_gather(t8_v, [(idx >> 3) & 7])
                                 + plsc.load_gather(t8_v, [idx & 7]))

    for b in range(17):
        idx = iota + b * L
        l = idx // 17
        sidx = jnp.minimum(idx - l * 17, 7)
        t8r_v[pl.ds(b * L, L)] = plsc.load_gather(t8_v, [sidx])

    def build(v, carry):
        b0 = v * (8 * L)
        for u in range(8):
            idx = iota + (b0 + u * L)
            t4_v[pl.ds(b0 + u * L, L)] = (plsc.load_gather(t2_v, [idx >> 6])
                                          + plsc.load_gather(t2_v, [idx & 63]))
        return carry

    lax.fori_loop(0, 4096 // (8 * L), build, 0)

    for k in range(NCHUNKS):
        buf = bufs[k % 2]
        pending[k % 2].wait()

        def supergroup(g, carry, buf=buf, k=k):
            srow = k * CHUNK_ROWS + g * SG_ROWS
            rbase = (g & 1) * 256
            for r in range(SG_ROWS):
                rvec = jnp.full((L,), g * SG_ROWS + r, jnp.int32)
                acc = None
                for q in range(3):
                    w0 = plsc.load_gather(buf, [rvec, iota + q * 64])
                    w1 = plsc.load_gather(buf, [rvec, iota + q * 64 + 16])
                    w2 = plsc.load_gather(buf, [rvec, iota + q * 64 + 32])
                    w3 = plsc.load_gather(buf, [rvec, iota + q * 64 + 48])
                    pidx = (w0 << 9) | (w1 << 6) | (w2 << 3) | w3
                    val = plsc.load_gather(t4_v, [pidx])
                    acc = val if acc is None else acc + val
                m = plsc.load_gather(buf, [rvec, iota + 184]) & 7
                mval = plsc.load_gather(t8r_v, [i17 + m])
                acc = acc + jnp.where(sel_lo, zf, mval)
                red_v[pl.ds(rbase + r * L, L)] = acc

            ra = en_v[pl.ds(srow, L)]
            rb2 = zf
            for cc in range(16):
                dix = rbase + i16 + ((iota + cc) & 15)
                v0 = plsc.load_gather(red_v, [dix])
                if cc % 2 == 0:
                    ra = ra + v0
                else:
                    rb2 = rb2 + v0
            out_v[pl.ds(srow, L)] = ra + rb2
            return carry

        lax.fori_loop(0, SGS_PER_CHUNK, supergroup, 0)

        if k + 2 < NCHUNKS:
            pending[k % 2] = start(k + 2)

    pltpu.sync_copy(out_v, out_hbm.at[pl.ds(base, ROWS_PER_TILE)])


def _make_sae():
    mesh = plsc.VectorSubcoreMesh(core_axis_name="c", subcore_axis_name="s")
    return functools.partial(
        pl.kernel,
        mesh=mesh,
        compiler_params=pltpu.CompilerParams(needs_layout_passes=False),
        out_type=jax.ShapeDtypeStruct((CONF,), jnp.float32),
        scratch_types=[
            pltpu.VMEM((CHUNK_ROWS, ATOMS), jnp.int32),
            pltpu.VMEM((CHUNK_ROWS, ATOMS), jnp.int32),
            pltpu.VMEM((8,), jnp.float32),
            pltpu.VMEM((17 * L,), jnp.float32),
            pltpu.VMEM((64,), jnp.float32),
            pltpu.VMEM((4096,), jnp.float32),
            pltpu.VMEM((2 * 256,), jnp.float32),
            pltpu.VMEM((ROWS_PER_TILE,), jnp.float32),
            pltpu.VMEM((ROWS_PER_TILE,), jnp.float32),
            pltpu.SemaphoreType.DMA,
            pltpu.SemaphoreType.DMA,
        ],
    )(_sae_body)


def kernel(species, energies, self_energies):
    spec_flat = species.astype(jnp.int32)
    se8 = jnp.zeros((8,), jnp.float32).at[:7].set(self_energies.astype(jnp.float32))
    out = _make_sae()(spec_flat, energies.astype(jnp.float32), se8)
    return (species, out)
